# baseline (speedup 1.0000x reference)
import jax
import jax.numpy as jnp
from jax.experimental import pallas as pl
from jax.experimental.pallas import tpu as pltpu

_N = 10
_NP = 16
_E = 90
_EP = 128
_TILE = 256
_NSTREAM = 4
_CI = (512, 1024, 1024, 2048, 2048, 256)
_CO = (1024, 1024, 2048, 2048, 256, 256)
_NT = tuple(c // _TILE for c in _CO)
_NS = (1, 1, 2, 2)
_S2 = (0, 1, 2, 4)
_STEPS = 7
_PREC = jax.lax.Precision.HIGHEST
_PREC_REF = jax.lax.Precision.DEFAULT


def _attention_matrix(sacc, dacc, e_ref):
    ones_row = jnp.ones((1, _TILE), jnp.float32)
    asrc_row = jax.lax.dot_general(
        ones_row, sacc, (((1,), (1,)), ((), ())),
        preferred_element_type=jnp.float32, precision=_PREC)
    adst_col = jnp.sum(dacc, axis=1, keepdims=True)
    a = adst_col + asrc_row
    a = jnp.where(a > 0, a, 0.2 * a)
    srow = e_ref[0:1, :]
    drow = e_ref[1:2, :]
    niota = jax.lax.broadcasted_iota(jnp.int32, (_NP, _EP), 0)
    oh_s = (srow == niota).astype(jnp.float32)
    oh_d = (drow == niota).astype(jnp.float32)
    cnt = jax.lax.dot_general(
        oh_d, oh_s, (((1,), (1,)), ((), ())),
        preferred_element_type=jnp.float32, precision=_PREC)
    r = jax.lax.broadcasted_iota(jnp.int32, (_NP, _NP), 0)
    c = jax.lax.broadcasted_iota(jnp.int32, (_NP, _NP), 1)
    cnt = cnt + ((r == c) & (r < _N)).astype(jnp.float32)
    has = cnt > 0
    am = jnp.max(jnp.where(has, a, -1e30), axis=1, keepdims=True)
    has_any = jnp.sum(cnt, axis=1, keepdims=True) > 0
    am = jnp.where(has_any, am, 0.0)
    ex = jnp.where(has, jnp.exp(a - am), 0.0) * cnt
    den = jnp.sum(ex, axis=1, keepdims=True)
    return ex / (den + 1e-16)


def _gat_tail(h, a_s, a_d, b_row, e_ref):
    att = _attention_matrix(h * a_s, h * a_d, e_ref)
    out = jax.lax.dot_general(
        att, h, (((1,), (0,)), ((), ())),
        preferred_element_type=jnp.float32, precision=_PREC)
    return jnp.maximum(out + b_row, 0.0)


def _body(*refs):
    (x_ref, e_ref) = refs[0:2]
    w = refs[2:18]
    w4, w5 = refs[18:20]
    pv = refs[20:38]
    heads_ref = refs[38]
    out_ref = refs[39]
    xb, hbuf, sacc, dacc, bb = refs[40:45]

    t = pl.program_id(0)
    for i in range(4):
        @pl.when((t >= _S2[i]) & (t < _S2[i] + _NS[i]))
        def _(i=i):
            j = t - _S2[i]
            if i == 0:
                xin = x_ref[...]
            else:
                xin = xb[:, : _CI[i]]
            a_s, a_d, b = pv[3 * i], pv[3 * i + 1], pv[3 * i + 2]
            sc = None
            dc = None
            for k in range(_NSTREAM):
                hk = jax.lax.dot_general(
                    xin, w[_NSTREAM * i + k][0], (((1,), (1,)), ((), ())),
                    preferred_element_type=jnp.float32,
                    precision=_PREC_REF)
                tix = _NSTREAM * j + k
                as_row = a_s[pl.ds(tix, 1), :]
                ad_row = a_d[pl.ds(tix, 1), :]
                sck = hk * as_row
                dck = hk * ad_row
                sc = sck if sc is None else sc + sck
                dc = dck if dc is None else dc + dck
                hbuf[:, pl.ds(tix * _TILE, _TILE)] = hk
                bb[0:1, pl.ds(tix * _TILE, _TILE)] = b[pl.ds(tix, 1), :]
            first = j == 0
            sacc[...] = jnp.where(first, sc, sacc[...] + sc)
            dacc[...] = jnp.where(first, dc, dacc[...] + dc)

            @pl.when(t == _S2[i] + _NS[i] - 1)
            def _():
                att = _attention_matrix(sacc[...], dacc[...], e_ref)
                hfull = hbuf[:, : _CO[i]]
                out = jax.lax.dot_general(
                    att, hfull, (((1,), (0,)), ((), ())),
                    preferred_element_type=jnp.float32, precision=_PREC)
                out = jnp.maximum(out + bb[0:1, : _CO[i]], 0.0)
                xb[:, : _CO[i]] = out

    @pl.when(t == _STEPS - 1)
    def _():
        h4 = jax.lax.dot_general(
            xb[:, : _CI[4]], w4[...], (((1,), (1,)), ((), ())),
            preferred_element_type=jnp.float32, precision=_PREC_REF)
        out4 = _gat_tail(h4, pv[12][...], pv[13][...], pv[14][...], e_ref)
        h5 = jax.lax.dot_general(
            out4, w5[...], (((1,), (1,)), ((), ())),
            preferred_element_type=jnp.float32, precision=_PREC_REF)
        out5 = _gat_tail(h5, pv[15][...], pv[16][...], pv[17][...], e_ref)
        lin_row = heads_ref[0:1, :_NP]
        z = jax.lax.dot_general(
            lin_row, out5, (((1,), (0,)), ((), ())),
            preferred_element_type=jnp.float32,
            precision=_PREC_REF) + heads_ref[3:4, 0:1]

        def _bf(u):
            return u.astype(jnp.bfloat16).astype(jnp.float32)
        v = (jnp.sum(_bf(jnp.tanh(z)) * _bf(heads_ref[1:2, :]),
                     axis=1, keepdims=True)
             + heads_ref[3:4, 1:2])
        cc = (jnp.sum(_bf(jnp.maximum(z, 0.0)) * _bf(heads_ref[2:3, :]),
                      axis=1, keepdims=True)
              + heads_ref[3:4, 2:3])
        cc = jax.nn.sigmoid(cc)
        ri = jax.lax.broadcasted_iota(jnp.int32, (8, 128), 0)
        ci = jax.lax.broadcasted_iota(jnp.int32, (8, 128), 1)
        res = jnp.where((ri == 0) & (ci == 0), v, 0.0)
        res = res + jnp.where((ri == 0) & (ci == 1), cc, 0.0)
        out_ref[...] = res


def kernel(x, edge_index, edge_attr, params):
    del edge_attr
    p = params
    names = ('g1a', 'g1b', 'g2a', 'g2b', 'g3a', 'g3b')
    ws = [p[n + '_W'].reshape(_NT[i], _TILE, _CI[i])
          for i, n in enumerate(names[:4])]
    pv_args = []
    for i, n in enumerate(names):
        pv_args.append(p[n + '_as'].reshape(_NT[i], _TILE))
        pv_args.append(p[n + '_ad'].reshape(_NT[i], _TILE))
        pv_args.append(p[n + '_b'].reshape(_NT[i], _TILE))

    lin_row = jnp.concatenate(
        [p['lin_W'][0], jnp.zeros((_TILE - _N,), jnp.float32)])[None, :]
    scal = jnp.concatenate(
        [p['lin_b'], p['reg_b'], p['cls_b'],
         jnp.zeros((_TILE - 3,), jnp.float32)])[None, :]
    heads = jnp.concatenate([lin_row, p['reg_W'], p['cls_W'], scal], axis=0)
    xp = jnp.pad(x, ((0, _NP - _N), (0, 0)))
    ep = jnp.pad(edge_index, ((0, 0), (0, _EP - _E)), constant_values=-1)

    in_specs = [
        pl.BlockSpec((_NP, _CI[0]), lambda t: (0, 0)),
        pl.BlockSpec((2, _EP), lambda t: (0, 0)),
    ]
    w_args = []
    for i in range(4):
        for k in range(_NSTREAM):
            in_specs.append(pl.BlockSpec(
                (1, _TILE, _CI[i]),
                lambda t, i=i, k=k: (
                    jnp.clip(_NSTREAM * (t - _S2[i]) + k, k,
                             _NT[i] - _NSTREAM + k), 0, 0)))
            w_args.append(ws[i])
    in_specs.append(pl.BlockSpec((_TILE, _CI[4]), lambda t: (0, 0)))
    in_specs.append(pl.BlockSpec((_TILE, _CI[5]), lambda t: (0, 0)))
    w_args += [p['g3a_W'], p['g3b_W']]
    for i in range(6):
        for _ in range(3):
            in_specs.append(pl.BlockSpec(
                (_NT[i], _TILE), lambda t: (0, 0)))
    in_specs.append(pl.BlockSpec((4, _TILE), lambda t: (0, 0)))

    out = pl.pallas_call(
        _body,
        grid=(_STEPS,),
        in_specs=in_specs,
        out_specs=pl.BlockSpec((8, 128), lambda t: (0, 0)),
        out_shape=jax.ShapeDtypeStruct((8, 128), jnp.float32),
        scratch_shapes=[
            pltpu.VMEM((_NP, 2048), jnp.float32),
            pltpu.VMEM((_NP, 2048), jnp.float32),
            pltpu.VMEM((_NP, _TILE), jnp.float32),
            pltpu.VMEM((_NP, _TILE), jnp.float32),
            pltpu.VMEM((8, 2048), jnp.float32),
        ],
        compiler_params=pltpu.CompilerParams(
            dimension_semantics=("arbitrary",)),
    )(xp, ep, *w_args, *pv_args, heads)
    return (out[0, 0:1], out[0, 1:2])


# speedup vs baseline: 1.0620x; 1.0620x over previous
import jax
import jax.numpy as jnp
from jax.experimental import pallas as pl
from jax.experimental.pallas import tpu as pltpu

_N = 10
_NP = 16
_E = 90
_EP = 128
_TILE = 256
_CI = (512, 1024, 1024, 2048, 2048, 256)
_CO = (1024, 1024, 2048, 2048, 256, 256)
_NT = tuple(c // _TILE for c in _CO)
_TILES = tuple((i, j) for i in range(6) for j in range(_NT[i]))
_DEPTH = 8
_CMAX = 2048
_PREC = jax.lax.Precision.HIGHEST
_PREC_REF = jax.lax.Precision.DEFAULT


def _attention_matrix(sacc, dacc, e_ref):
    ones_row = jnp.ones((1, _TILE), jnp.float32)
    asrc_row = jax.lax.dot_general(
        ones_row, sacc, (((1,), (1,)), ((), ())),
        preferred_element_type=jnp.float32, precision=_PREC)
    adst_col = jnp.sum(dacc, axis=1, keepdims=True)
    a = adst_col + asrc_row
    a = jnp.where(a > 0, a, 0.2 * a)
    srow = e_ref[0:1, :]
    drow = e_ref[1:2, :]
    niota = jax.lax.broadcasted_iota(jnp.int32, (_NP, _EP), 0)
    oh_s = (srow == niota).astype(jnp.float32)
    oh_d = (drow == niota).astype(jnp.float32)
    cnt = jax.lax.dot_general(
        oh_d, oh_s, (((1,), (1,)), ((), ())),
        preferred_element_type=jnp.float32, precision=_PREC)
    r = jax.lax.broadcasted_iota(jnp.int32, (_NP, _NP), 0)
    c = jax.lax.broadcasted_iota(jnp.int32, (_NP, _NP), 1)
    cnt = cnt + ((r == c) & (r < _N)).astype(jnp.float32)
    has = cnt > 0
    am = jnp.max(jnp.where(has, a, -1e30), axis=1, keepdims=True)
    has_any = jnp.sum(cnt, axis=1, keepdims=True) > 0
    am = jnp.where(has_any, am, 0.0)
    ex = jnp.where(has, jnp.exp(a - am), 0.0) * cnt
    den = jnp.sum(ex, axis=1, keepdims=True)
    return ex / (den + 1e-16)


def _body(*refs):
    (x_ref, e_ref) = refs[0:2]
    w = refs[2:8]
    pv = refs[8:26]
    heads_ref = refs[26]
    out_ref = refs[27]
    xb, hbuf, bb, vbuf, sem = refs[28:33]

    def copy_start(n, slot):
        i, j = _TILES[n]
        pltpu.make_async_copy(
            w[i].at[j * _TILE:(j + 1) * _TILE, :],
            vbuf.at[slot, :, : _CI[i]],
            sem.at[slot]).start()

    for n in range(_DEPTH):
        copy_start(n, n)

    sacc = None
    dacc = None
    for n, (i, j) in enumerate(_TILES):
        slot = n % _DEPTH
        ci, co = _CI[i], _CO[i]
        pltpu.make_async_copy(
            w[i].at[j * _TILE:(j + 1) * _TILE, :],
            vbuf.at[slot, :, :ci],
            sem.at[slot]).wait()
        if i == 0:
            xin = x_ref[...]
        else:
            xin = xb[:, :ci]
        h = jax.lax.dot_general(
            xin, vbuf[slot, :, :ci], (((1,), (1,)), ((), ())),
            preferred_element_type=jnp.float32, precision=_PREC_REF)
        if n + _DEPTH < len(_TILES):
            copy_start(n + _DEPTH, slot)
        a_s, a_d, b = pv[3 * i], pv[3 * i + 1], pv[3 * i + 2]
        sck = h * a_s[j:j + 1, :]
        dck = h * a_d[j:j + 1, :]
        sacc = sck if j == 0 else sacc + sck
        dacc = dck if j == 0 else dacc + dck
        hbuf[:, j * _TILE:(j + 1) * _TILE] = h
        bb[0:1, j * _TILE:(j + 1) * _TILE] = b[j:j + 1, :]
        if j == _NT[i] - 1:
            att = _attention_matrix(sacc, dacc, e_ref)
            hfull = hbuf[:, :co]
            out = jax.lax.dot_general(
                att, hfull, (((1,), (0,)), ((), ())),
                preferred_element_type=jnp.float32, precision=_PREC)
            out = jnp.maximum(out + bb[0:1, :co], 0.0)
            xb[:, :co] = out
            if i == 5:
                lin_row = heads_ref[0:1, :_NP]
                z = jax.lax.dot_general(
                    lin_row, out, (((1,), (0,)), ((), ())),
                    preferred_element_type=jnp.float32,
                    precision=_PREC_REF) + heads_ref[3:4, 0:1]

                def _bf(u):
                    return u.astype(jnp.bfloat16).astype(jnp.float32)
                v = (jnp.sum(_bf(jnp.tanh(z)) * _bf(heads_ref[1:2, :]),
                             axis=1, keepdims=True)
                     + heads_ref[3:4, 1:2])
                cc = (jnp.sum(_bf(jnp.maximum(z, 0.0))
                              * _bf(heads_ref[2:3, :]),
                              axis=1, keepdims=True)
                      + heads_ref[3:4, 2:3])
                cc = jax.nn.sigmoid(cc)
                ri = jax.lax.broadcasted_iota(jnp.int32, (8, 128), 0)
                cidx = jax.lax.broadcasted_iota(jnp.int32, (8, 128), 1)
                res = jnp.where((ri == 0) & (cidx == 0), v, 0.0)
                res = res + jnp.where((ri == 0) & (cidx == 1), cc, 0.0)
                out_ref[...] = res


def kernel(x, edge_index, edge_attr, params):
    del edge_attr
    p = params
    names = ('g1a', 'g1b', 'g2a', 'g2b', 'g3a', 'g3b')
    w_args = [p[n + '_W'] for n in names]
    pv_args = []
    for i, n in enumerate(names):
        pv_args.append(p[n + '_as'].reshape(_NT[i], _TILE))
        pv_args.append(p[n + '_ad'].reshape(_NT[i], _TILE))
        pv_args.append(p[n + '_b'].reshape(_NT[i], _TILE))

    lin_row = jnp.concatenate(
        [p['lin_W'][0], jnp.zeros((_TILE - _N,), jnp.float32)])[None, :]
    scal = jnp.concatenate(
        [p['lin_b'], p['reg_b'], p['cls_b'],
         jnp.zeros((_TILE - 3,), jnp.float32)])[None, :]
    heads = jnp.concatenate([lin_row, p['reg_W'], p['cls_W'], scal], axis=0)
    xp = jnp.pad(x, ((0, _NP - _N), (0, 0)))
    ep = jnp.pad(edge_index, ((0, 0), (0, _EP - _E)), constant_values=-1)

    in_specs = [
        pl.BlockSpec((_NP, _CI[0]), lambda: (0, 0)),
        pl.BlockSpec((2, _EP), lambda: (0, 0)),
    ]
    for i in range(6):
        in_specs.append(pl.BlockSpec(memory_space=pl.ANY))
    for i in range(6):
        for _ in range(3):
            in_specs.append(pl.BlockSpec(
                (_NT[i], _TILE), lambda i=i: (0, 0)))
    in_specs.append(pl.BlockSpec((4, _TILE), lambda: (0, 0)))

    out = pl.pallas_call(
        _body,
        in_specs=in_specs,
        out_specs=pl.BlockSpec((8, 128), lambda: (0, 0)),
        out_shape=jax.ShapeDtypeStruct((8, 128), jnp.float32),
        scratch_shapes=[
            pltpu.VMEM((_NP, _CMAX), jnp.float32),
            pltpu.VMEM((_NP, _CMAX), jnp.float32),
            pltpu.VMEM((8, _CMAX), jnp.float32),
            pltpu.VMEM((_DEPTH, _TILE, _CMAX), jnp.float32),
            pltpu.SemaphoreType.DMA((_DEPTH,)),
        ],
    )(xp, ep, *w_args, *pv_args, heads)
    return (out[0, 0:1], out[0, 1:2])


# speedup vs baseline: 1.4868x; 1.4000x over previous
import jax
import jax.numpy as jnp
from jax.experimental import pallas as pl
from jax.experimental.pallas import tpu as pltpu

_N = 10
_NP = 16
_E = 90
_EP = 128
_TILE = 256
_CI = (512, 1024, 1024, 2048, 2048, 256)
_CO = (1024, 1024, 2048, 2048, 256, 256)
_NT = tuple(c // _TILE for c in _CO)
_NS = (2, 2, 4, 4)
_S2 = (0, 2, 4, 8)
_STEPS = 13
_PREC = jax.lax.Precision.HIGHEST
_PREC_REF = jax.lax.Precision.DEFAULT

_AS = []
_AD = []
_B = []
_base = 0
for _i in range(6):
    _AS.append(_base)
    _AD.append(_base + _NT[_i])
    _B.append(_base + 2 * _NT[_i])
    _base += 3 * _NT[_i]
_ROW_REG = _base
_ROW_CLS = _base + 1
_ROW_LIN = _base + 2
_ROW_SC = _base + 3
_TBL_ROWS = _base + 4


def _attention_matrix(sacc, dacc, e_ref):
    ones_row = jnp.ones((1, _TILE), jnp.float32)
    asrc_row = jax.lax.dot_general(
        ones_row, sacc, (((1,), (1,)), ((), ())),
        preferred_element_type=jnp.float32, precision=_PREC)
    adst_col = jnp.sum(dacc, axis=1, keepdims=True)
    a = adst_col + asrc_row
    a = jnp.where(a > 0, a, 0.2 * a)
    srow = e_ref[0:1, :]
    drow = e_ref[1:2, :]
    niota = jax.lax.broadcasted_iota(jnp.int32, (_NP, _EP), 0)
    oh_s = (srow == niota).astype(jnp.float32)
    oh_d = (drow == niota).astype(jnp.float32)
    cnt = jax.lax.dot_general(
        oh_d, oh_s, (((1,), (1,)), ((), ())),
        preferred_element_type=jnp.float32, precision=_PREC)
    r = jax.lax.broadcasted_iota(jnp.int32, (_NP, _NP), 0)
    c = jax.lax.broadcasted_iota(jnp.int32, (_NP, _NP), 1)
    cnt = cnt + ((r == c) & (r < _N)).astype(jnp.float32)
    has = cnt > 0
    am = jnp.max(jnp.where(has, a, -1e30), axis=1, keepdims=True)
    has_any = jnp.sum(cnt, axis=1, keepdims=True) > 0
    am = jnp.where(has_any, am, 0.0)
    ex = jnp.where(has, jnp.exp(a - am), 0.0) * cnt
    den = jnp.sum(ex, axis=1, keepdims=True)
    return ex / (den + 1e-16)


def _gat_tail(h, a_s, a_d, b_row, e_ref):
    att = _attention_matrix(h * a_s, h * a_d, e_ref)
    out = jax.lax.dot_general(
        att, h, (((1,), (0,)), ((), ())),
        preferred_element_type=jnp.float32, precision=_PREC)
    return jnp.maximum(out + b_row, 0.0)


def _body(x_ref, e_ref, wa0, wb0, wa1, wb1, wa2, wb2, wa3, wb3, w4, w5,
          tbl_ref, out_ref, xb, hbuf, sacc, dacc, bb):
    t = pl.program_id(0)
    wa = (wa0, wa1, wa2, wa3)
    wb = (wb0, wb1, wb2, wb3)
    for i in range(4):
        @pl.when((t >= _S2[i]) & (t < _S2[i] + _NS[i]))
        def _(i=i):
            j = t - _S2[i]
            if i == 0:
                xin = x_ref[...]
            else:
                xin = xb[:, : _CI[i]]
            ha = jax.lax.dot_general(
                xin, wa[i][0], (((1,), (1,)), ((), ())),
                preferred_element_type=jnp.float32, precision=_PREC_REF)
            hc = jax.lax.dot_general(
                xin, wb[i][0], (((1,), (1,)), ((), ())),
                preferred_element_type=jnp.float32, precision=_PREC_REF)
            sc = (ha * tbl_ref[pl.ds(_AS[i] + 2 * j, 1), :]
                  + hc * tbl_ref[pl.ds(_AS[i] + 2 * j + 1, 1), :])
            dc = (ha * tbl_ref[pl.ds(_AD[i] + 2 * j, 1), :]
                  + hc * tbl_ref[pl.ds(_AD[i] + 2 * j + 1, 1), :])
            first = j == 0
            sacc[...] = jnp.where(first, sc, sacc[...] + sc)
            dacc[...] = jnp.where(first, dc, dacc[...] + dc)
            hbuf[:, pl.ds((2 * j) * _TILE, _TILE)] = ha
            hbuf[:, pl.ds((2 * j + 1) * _TILE, _TILE)] = hc
            bb[0:1, pl.ds((2 * j) * _TILE, _TILE)] = \
                tbl_ref[pl.ds(_B[i] + 2 * j, 1), :]
            bb[0:1, pl.ds((2 * j + 1) * _TILE, _TILE)] = \
                tbl_ref[pl.ds(_B[i] + 2 * j + 1, 1), :]

            @pl.when(t == _S2[i] + _NS[i] - 1)
            def _():
                att = _attention_matrix(sacc[...], dacc[...], e_ref)
                hfull = hbuf[:, : _CO[i]]
                out = jax.lax.dot_general(
                    att, hfull, (((1,), (0,)), ((), ())),
                    preferred_element_type=jnp.float32, precision=_PREC)
                out = jnp.maximum(out + bb[0:1, : _CO[i]], 0.0)
                xb[:, : _CO[i]] = out

    @pl.when(t == _STEPS - 1)
    def _():
        h4 = jax.lax.dot_general(
            xb[:, : _CI[4]], w4[...], (((1,), (1,)), ((), ())),
            preferred_element_type=jnp.float32, precision=_PREC_REF)
        out4 = _gat_tail(h4, tbl_ref[_AS[4]:_AS[4] + 1, :],
                         tbl_ref[_AD[4]:_AD[4] + 1, :],
                         tbl_ref[_B[4]:_B[4] + 1, :], e_ref)
        h5 = jax.lax.dot_general(
            out4, w5[...], (((1,), (1,)), ((), ())),
            preferred_element_type=jnp.float32, precision=_PREC_REF)
        out5 = _gat_tail(h5, tbl_ref[_AS[5]:_AS[5] + 1, :],
                         tbl_ref[_AD[5]:_AD[5] + 1, :],
                         tbl_ref[_B[5]:_B[5] + 1, :], e_ref)
        lin_row = tbl_ref[_ROW_LIN:_ROW_LIN + 1, :_NP]
        z = jax.lax.dot_general(
            lin_row, out5, (((1,), (0,)), ((), ())),
            preferred_element_type=jnp.float32,
            precision=_PREC_REF) + tbl_ref[_ROW_SC:_ROW_SC + 1, 0:1]

        def _bf(u):
            return u.astype(jnp.bfloat16).astype(jnp.float32)
        v = (jnp.sum(_bf(jnp.tanh(z))
                     * _bf(tbl_ref[_ROW_REG:_ROW_REG + 1, :]),
                     axis=1, keepdims=True)
             + tbl_ref[_ROW_SC:_ROW_SC + 1, 1:2])
        cc = (jnp.sum(_bf(jnp.maximum(z, 0.0))
                      * _bf(tbl_ref[_ROW_CLS:_ROW_CLS + 1, :]),
                      axis=1, keepdims=True)
              + tbl_ref[_ROW_SC:_ROW_SC + 1, 2:3])
        cc = jax.nn.sigmoid(cc)
        ri = jax.lax.broadcasted_iota(jnp.int32, (8, 128), 0)
        ci = jax.lax.broadcasted_iota(jnp.int32, (8, 128), 1)
        res = jnp.where((ri == 0) & (ci == 0), v, 0.0)
        res = res + jnp.where((ri == 0) & (ci == 1), cc, 0.0)
        out_ref[...] = res


def kernel(x, edge_index, edge_attr, params):
    del edge_attr
    p = params
    names = ('g1a', 'g1b', 'g2a', 'g2b', 'g3a', 'g3b')
    ws = [p[n + '_W'].reshape(_NT[i], _TILE, _CI[i])
          for i, n in enumerate(names)]

    pieces = []
    for n in names:
        pieces += [p[n + '_as'], p[n + '_ad'], p[n + '_b']]
    pieces += [p['reg_W'][0], p['cls_W'][0],
               p['lin_W'][0], jnp.zeros((_TILE - _N,), jnp.float32),
               p['lin_b'], p['reg_b'], p['cls_b'],
               jnp.zeros((_TILE - 3,), jnp.float32)]
    tbl = jnp.concatenate(pieces).reshape(_TBL_ROWS, _TILE)
    xp = jnp.pad(x, ((0, _NP - _N), (0, 0)))
    ep = jnp.pad(edge_index, ((0, 0), (0, _EP - _E)), constant_values=-1)

    in_specs = [
        pl.BlockSpec((_NP, _CI[0]), lambda t: (0, 0)),
        pl.BlockSpec((2, _EP), lambda t: (0, 0)),
    ]
    for i in range(4):
        in_specs.append(pl.BlockSpec(
            (1, _TILE, _CI[i]),
            lambda t, i=i: (jnp.clip(2 * (t - _S2[i]), 0, _NT[i] - 2), 0, 0)))
        in_specs.append(pl.BlockSpec(
            (1, _TILE, _CI[i]),
            lambda t, i=i: (jnp.clip(2 * (t - _S2[i]) + 1, 1, _NT[i] - 1),
                            0, 0)))
    in_specs.append(pl.BlockSpec((_TILE, _CI[4]), lambda t: (0, 0)))
    in_specs.append(pl.BlockSpec((_TILE, _CI[5]), lambda t: (0, 0)))
    in_specs.append(pl.BlockSpec((_TBL_ROWS, _TILE), lambda t: (0, 0)))

    w_args = []
    for i in range(4):
        w_args += [ws[i], ws[i]]
    w_args += [p['g3a_W'], p['g3b_W']]

    out = pl.pallas_call(
        _body,
        grid=(_STEPS,),
        in_specs=in_specs,
        out_specs=pl.BlockSpec((8, 128), lambda t: (0, 0)),
        out_shape=jax.ShapeDtypeStruct((8, 128), jnp.float32),
        scratch_shapes=[
            pltpu.VMEM((_NP, 2048), jnp.float32),
            pltpu.VMEM((_NP, 2048), jnp.float32),
            pltpu.VMEM((_NP, _TILE), jnp.float32),
            pltpu.VMEM((_NP, _TILE), jnp.float32),
            pltpu.VMEM((8, 2048), jnp.float32),
        ],
        compiler_params=pltpu.CompilerParams(
            dimension_semantics=("arbitrary",)),
    )(xp, ep, *w_args, tbl)
    return (out[0, 0:1], out[0, 1:2])


# speedup vs baseline: 1.6331x; 1.0984x over previous
import jax
import jax.numpy as jnp
from jax.experimental import pallas as pl
from jax.experimental.pallas import tpu as pltpu

_N = 10
_NP = 16
_E = 90
_EP = 128
_TILE = 256
_CI = (512, 1024, 1024, 2048, 2048, 256)
_CO = (1024, 1024, 2048, 2048, 256, 256)
_NT = tuple(c // _TILE for c in _CO)
_NS = (2, 2, 4, 4)
_S2 = (0, 2, 4, 8)
_STEPS = 13
_PREC = jax.lax.Precision.HIGHEST
_PREC_REF = jax.lax.Precision.DEFAULT

_AS = []
_AD = []
_B = []
_base = 0
for _i in range(6):
    _AS.append(_base)
    _AD.append(_base + _NT[_i])
    _B.append(_base + 2 * _NT[_i])
    _base += 3 * _NT[_i]
_ROW_REG = _base
_ROW_CLS = _base + 1
_ROW_LIN = _base + 2
_ROW_SC = _base + 3
_TBL_ROWS = _base + 4


def _attention_matrix(sacc, dacc, e_ref):
    ones_row = jnp.ones((1, _TILE), jnp.float32)
    asrc_row = jax.lax.dot_general(
        ones_row, sacc, (((1,), (1,)), ((), ())),
        preferred_element_type=jnp.float32, precision=_PREC)
    adst_col = jnp.sum(dacc, axis=1, keepdims=True)
    a = adst_col + asrc_row
    a = jnp.where(a > 0, a, 0.2 * a)
    srow = e_ref[0:1, :]
    drow = e_ref[1:2, :]
    niota = jax.lax.broadcasted_iota(jnp.int32, (_NP, _E), 0)
    oh_s = (srow == niota).astype(jnp.float32)
    oh_d = (drow == niota).astype(jnp.float32)
    cnt = jax.lax.dot_general(
        oh_d, oh_s, (((1,), (1,)), ((), ())),
        preferred_element_type=jnp.float32, precision=_PREC)
    r = jax.lax.broadcasted_iota(jnp.int32, (_NP, _NP), 0)
    c = jax.lax.broadcasted_iota(jnp.int32, (_NP, _NP), 1)
    cnt = cnt + ((r == c) & (r < _N)).astype(jnp.float32)
    has = cnt > 0
    am = jnp.max(jnp.where(has, a, -1e30), axis=1, keepdims=True)
    has_any = jnp.sum(cnt, axis=1, keepdims=True) > 0
    am = jnp.where(has_any, am, 0.0)
    ex = jnp.where(has, jnp.exp(a - am), 0.0) * cnt
    den = jnp.sum(ex, axis=1, keepdims=True)
    return ex / (den + 1e-16)


def _gat_tail(h, a_s, a_d, b_row, e_ref):
    att = _attention_matrix(h * a_s, h * a_d, e_ref)
    out = jax.lax.dot_general(
        att, h, (((1,), (0,)), ((), ())),
        preferred_element_type=jnp.float32, precision=_PREC)
    return jnp.maximum(out + b_row, 0.0)


def _body(x_ref, e_ref, wa0, wb0, wa1, wb1, wa2, wb2, wa3, wb3, w4, w5,
          tbl_ref, out_ref, xb, hbuf, sacc, dacc, bb):
    t = pl.program_id(0)
    wa = (wa0, wa1, wa2, wa3)
    wb = (wb0, wb1, wb2, wb3)
    for i in range(4):
        @pl.when((t >= _S2[i]) & (t < _S2[i] + _NS[i]))
        def _(i=i):
            j = t - _S2[i]
            if i == 0:
                xin = x_ref[...]
                nr = _N

                @pl.when(t == 0)
                def _():
                    hbuf[_N:, :] = jnp.zeros((_NP - _N, hbuf.shape[1]),
                                             jnp.float32)
            else:
                xin = xb[:, : _CI[i]]
                nr = _NP
            ha = jax.lax.dot_general(
                xin, wa[i][0], (((1,), (1,)), ((), ())),
                preferred_element_type=jnp.float32, precision=_PREC_REF)
            hc = jax.lax.dot_general(
                xin, wb[i][0], (((1,), (1,)), ((), ())),
                preferred_element_type=jnp.float32, precision=_PREC_REF)
            sc = (ha * tbl_ref[pl.ds(_AS[i] + 2 * j, 1), :]
                  + hc * tbl_ref[pl.ds(_AS[i] + 2 * j + 1, 1), :])
            dc = (ha * tbl_ref[pl.ds(_AD[i] + 2 * j, 1), :]
                  + hc * tbl_ref[pl.ds(_AD[i] + 2 * j + 1, 1), :])
            first = j == 0
            sacc[0:nr, :] = jnp.where(first, sc, sacc[0:nr, :] + sc)
            dacc[0:nr, :] = jnp.where(first, dc, dacc[0:nr, :] + dc)
            hbuf[0:nr, pl.ds((2 * j) * _TILE, _TILE)] = ha
            hbuf[0:nr, pl.ds((2 * j + 1) * _TILE, _TILE)] = hc
            bb[0:1, pl.ds((2 * j) * _TILE, _TILE)] = \
                tbl_ref[pl.ds(_B[i] + 2 * j, 1), :]
            bb[0:1, pl.ds((2 * j + 1) * _TILE, _TILE)] = \
                tbl_ref[pl.ds(_B[i] + 2 * j + 1, 1), :]

            @pl.when(t == _S2[i] + _NS[i] - 1)
            def _():
                att = _attention_matrix(sacc[...], dacc[...], e_ref)
                hfull = hbuf[:, : _CO[i]]
                out = jax.lax.dot_general(
                    att, hfull, (((1,), (0,)), ((), ())),
                    preferred_element_type=jnp.float32, precision=_PREC)
                out = jnp.maximum(out + bb[0:1, : _CO[i]], 0.0)
                xb[:, : _CO[i]] = out

    @pl.when(t == _STEPS - 1)
    def _():
        h4 = jax.lax.dot_general(
            xb[:, : _CI[4]], w4[...], (((1,), (1,)), ((), ())),
            preferred_element_type=jnp.float32, precision=_PREC_REF)
        out4 = _gat_tail(h4, tbl_ref[_AS[4]:_AS[4] + 1, :],
                         tbl_ref[_AD[4]:_AD[4] + 1, :],
                         tbl_ref[_B[4]:_B[4] + 1, :], e_ref)
        h5 = jax.lax.dot_general(
            out4, w5[...], (((1,), (1,)), ((), ())),
            preferred_element_type=jnp.float32, precision=_PREC_REF)
        out5 = _gat_tail(h5, tbl_ref[_AS[5]:_AS[5] + 1, :],
                         tbl_ref[_AD[5]:_AD[5] + 1, :],
                         tbl_ref[_B[5]:_B[5] + 1, :], e_ref)
        lin_row = tbl_ref[_ROW_LIN:_ROW_LIN + 1, :_NP]
        z = jax.lax.dot_general(
            lin_row, out5, (((1,), (0,)), ((), ())),
            preferred_element_type=jnp.float32,
            precision=_PREC_REF) + tbl_ref[_ROW_SC:_ROW_SC + 1, 0:1]

        def _bf(u):
            return u.astype(jnp.bfloat16).astype(jnp.float32)
        v = (jnp.sum(_bf(jnp.tanh(z))
                     * _bf(tbl_ref[_ROW_REG:_ROW_REG + 1, :]),
                     axis=1, keepdims=True)
             + tbl_ref[_ROW_SC:_ROW_SC + 1, 1:2])
        cc = (jnp.sum(_bf(jnp.maximum(z, 0.0))
                      * _bf(tbl_ref[_ROW_CLS:_ROW_CLS + 1, :]),
                      axis=1, keepdims=True)
              + tbl_ref[_ROW_SC:_ROW_SC + 1, 2:3])
        cc = jax.nn.sigmoid(cc)
        ri = jax.lax.broadcasted_iota(jnp.int32, (8, 128), 0)
        ci = jax.lax.broadcasted_iota(jnp.int32, (8, 128), 1)
        res = jnp.where((ri == 0) & (ci == 0), v, 0.0)
        res = res + jnp.where((ri == 0) & (ci == 1), cc, 0.0)
        out_ref[...] = res


def kernel(x, edge_index, edge_attr, params):
    del edge_attr
    p = params
    names = ('g1a', 'g1b', 'g2a', 'g2b', 'g3a', 'g3b')
    ws = [p[n + '_W'].reshape(_NT[i], _TILE, _CI[i])
          for i, n in enumerate(names)]

    pieces = []
    for n in names:
        pieces += [p[n + '_as'], p[n + '_ad'], p[n + '_b']]
    pieces += [p['reg_W'][0], p['cls_W'][0],
               p['lin_W'][0], jnp.zeros((_TILE - _N,), jnp.float32),
               p['lin_b'], p['reg_b'], p['cls_b'],
               jnp.zeros((_TILE - 3,), jnp.float32)]
    tbl = jnp.concatenate(pieces).reshape(_TBL_ROWS, _TILE)

    in_specs = [
        pl.BlockSpec((_N, _CI[0]), lambda t: (0, 0)),
        pl.BlockSpec((2, _E), lambda t: (0, 0)),
    ]
    for i in range(4):
        in_specs.append(pl.BlockSpec(
            (1, _TILE, _CI[i]),
            lambda t, i=i: (jnp.clip(2 * (t - _S2[i]), 0, _NT[i] - 2), 0, 0)))
        in_specs.append(pl.BlockSpec(
            (1, _TILE, _CI[i]),
            lambda t, i=i: (jnp.clip(2 * (t - _S2[i]) + 1, 1, _NT[i] - 1),
                            0, 0)))
    in_specs.append(pl.BlockSpec((_TILE, _CI[4]), lambda t: (0, 0)))
    in_specs.append(pl.BlockSpec((_TILE, _CI[5]), lambda t: (0, 0)))
    in_specs.append(pl.BlockSpec((_TBL_ROWS, _TILE), lambda t: (0, 0)))

    w_args = []
    for i in range(4):
        w_args += [ws[i], ws[i]]
    w_args += [p['g3a_W'], p['g3b_W']]

    out = pl.pallas_call(
        _body,
        grid=(_STEPS,),
        in_specs=in_specs,
        out_specs=pl.BlockSpec((8, 128), lambda t: (0, 0)),
        out_shape=jax.ShapeDtypeStruct((8, 128), jnp.float32),
        scratch_shapes=[
            pltpu.VMEM((_NP, 2048), jnp.float32),
            pltpu.VMEM((_NP, 2048), jnp.float32),
            pltpu.VMEM((_NP, _TILE), jnp.float32),
            pltpu.VMEM((_NP, _TILE), jnp.float32),
            pltpu.VMEM((8, 2048), jnp.float32),
        ],
        compiler_params=pltpu.CompilerParams(
            dimension_semantics=("arbitrary",)),
    )(x, edge_index, *w_args, tbl)
    return (out[0, 0:1], out[0, 1:2])
